# baseline (speedup 1.0000x reference)
import functools

import jax
import jax.numpy as jnp
from jax import lax
from jax.experimental import pallas as pl
from jax.experimental.pallas import tpu as pltpu
from jax.experimental.pallas import tpu_sc as plsc

_NC = 2
_NS = 16
_CH = 128
_NZ = 5
_NPAD = _NS * _NZ * _CH


def _sc_aggregate(feat, src1d, dst1d, zeros_hbm, ones_hbm, iota_hbm,
                  *, with_cnt):
    n_pad, d = feat.shape
    assert n_pad == _NPAD
    e_pad = src1d.shape[0] * _CH
    cpt = e_pad // (_NC * _NS * _CH)
    assert e_pad == cpt * _NC * _NS * _CH and cpt % 16 == 0
    hcpt = cpt // 2

    out_type = [jax.ShapeDtypeStruct((_NC, n_pad, d), jnp.float32)]
    if with_cnt:
        out_type.append(jax.ShapeDtypeStruct((_NC, n_pad), jnp.float32))

    scratch = [
        pltpu.VMEM((hcpt, _CH), jnp.int32),
        pltpu.VMEM((hcpt, _CH), jnp.int32),
        pltpu.VMEM((_CH,), jnp.int32),
        pltpu.VMEM((_CH, d), jnp.float32),
        pltpu.VMEM((_CH,), jnp.float32),
        pltpu.VMEM_SHARED((n_pad, d), jnp.float32),
        pltpu.SemaphoreType.DMA,
    ]
    if with_cnt:
        scratch.append(pltpu.VMEM_SHARED((n_pad,), jnp.float32))

    def body(feat_hbm, src_hbm, dst_hbm, z_hbm, o_hbm, iota_hbm,
             psum_hbm, *rest):
        if with_cnt:
            pcnt_hbm, src_a, dst_a, dst_c, rows_v, c_v, acc, sem, cacc = rest
        else:
            src_a, dst_a, dst_c, rows_v, c_v, acc, sem = rest

        cid = lax.axis_index("c")
        sid = lax.axis_index("s")
        wid = cid * _NS + sid

        pltpu.sync_copy(z_hbm, rows_v)
        pltpu.sync_copy(z_hbm.at[0], c_v)
        zbase = sid * (_NZ * _CH)
        for k in range(_NZ):
            pltpu.sync_copy(iota_hbm.at[pl.ds(zbase + k * _CH, _CH)], dst_c)
            pltpu.sync_copy(rows_v, acc.at[dst_c])
            if with_cnt:
                pltpu.sync_copy(c_v, cacc.at[dst_c])
        if with_cnt:
            pltpu.sync_copy(o_hbm, c_v)
        plsc.subcore_barrier()

        for half in range(2):
            rowbase = wid * cpt + half * hcpt
            pltpu.sync_copy(src_hbm.at[pl.ds(rowbase, hcpt)], src_a)
            pltpu.sync_copy(dst_hbm.at[pl.ds(rowbase, hcpt)], dst_a)

            def step(j, carry):
                pltpu.async_copy(feat_hbm.at[src_a.at[j]], rows_v, sem)
                for t in range(_CH // 16):
                    dst_c[pl.ds(t * 16, 16)] = dst_a[j, pl.ds(t * 16, 16)]
                pltpu.make_async_copy(
                    feat_hbm.at[src_a.at[j]], rows_v, sem).wait()
                pltpu.sync_copy(rows_v, acc.at[dst_c], add=True)
                if with_cnt:
                    pltpu.sync_copy(c_v, cacc.at[dst_c], add=True)
                return carry

            lax.fori_loop(0, hcpt, step, 0)
        plsc.subcore_barrier()

        for k in range(_NZ):
            base = zbase + k * _CH
            pltpu.sync_copy(iota_hbm.at[pl.ds(base, _CH)], dst_c)
            pltpu.async_copy(acc.at[dst_c], rows_v, sem).wait()
            pltpu.sync_copy(rows_v, psum_hbm.at[cid, pl.ds(base, _CH)])
            if with_cnt:
                pltpu.async_copy(cacc.at[dst_c], c_v, sem).wait()
                pltpu.sync_copy(c_v, pcnt_hbm.at[cid, pl.ds(base, _CH)])

    mesh = plsc.VectorSubcoreMesh(core_axis_name="c", subcore_axis_name="s")
    fn = pl.kernel(body, out_type=out_type, mesh=mesh, scratch_types=scratch)
    return fn(feat, src1d, dst1d, zeros_hbm, ones_hbm, iota_hbm)


def _dense_body(relu, psum_ref, cnt_ref, x_ref, wl_ref, wr_ref, b_ref, o_ref):
    s = psum_ref[0] + psum_ref[1]
    c = cnt_ref[0] + cnt_ref[1]
    agg = s * (1.0 / jnp.maximum(c, 1.0))
    y = jnp.dot(agg, wl_ref[...], preferred_element_type=jnp.float32)
    y = y + jnp.dot(x_ref[...], wr_ref[...], preferred_element_type=jnp.float32)
    y = y + b_ref[...]
    o_ref[...] = jnp.maximum(y, 0.0) if relu else y


def _dense(psum, pcnt3, x, wlT, wrT, b, *, relu, block_rows=512):
    n, d = x.shape
    hid = wlT.shape[1]
    assert n % block_rows == 0
    grid = (n // block_rows,)
    return pl.pallas_call(
        functools.partial(_dense_body, relu),
        grid=grid,
        in_specs=[
            pl.BlockSpec((_NC, block_rows, d), lambda i: (0, i, 0)),
            pl.BlockSpec((_NC, block_rows, 1), lambda i: (0, i, 0)),
            pl.BlockSpec((block_rows, d), lambda i: (i, 0)),
            pl.BlockSpec((d, hid), lambda i: (0, 0)),
            pl.BlockSpec((d, hid), lambda i: (0, 0)),
            pl.BlockSpec((1, hid), lambda i: (0, 0)),
        ],
        out_specs=pl.BlockSpec((block_rows, hid), lambda i: (i, 0)),
        out_shape=jax.ShapeDtypeStruct((n, hid), jnp.float32),
    )(psum, pcnt3, x, wlT, wrT, b)


def kernel(x, edge_index, W1l, b1l, W1r, W2l, b2l, W2r):
    n, d = x.shape
    e = edge_index.shape[1]
    assert n <= _NPAD

    e_pad = -(-e // (16 * _NC * _NS * _CH)) * (16 * _NC * _NS * _CH)
    src1d = jnp.concatenate(
        [edge_index[0], jnp.zeros((e_pad - e,), jnp.int32)]
    ).reshape(e_pad // _CH, _CH)
    dst1d = jnp.concatenate(
        [edge_index[1], jnp.full((e_pad - e,), n, jnp.int32)]
    ).reshape(e_pad // _CH, _CH)
    zeros = jnp.zeros((_CH, d), jnp.float32)
    ones = jnp.ones((_CH,), jnp.float32)
    iota = jnp.arange(_NPAD, dtype=jnp.int32)
    x_pad = jnp.concatenate(
        [x, jnp.zeros((_NPAD - n, d), jnp.float32)], axis=0)

    psum1, pcnt = _sc_aggregate(x_pad, src1d, dst1d, zeros, ones, iota,
                                with_cnt=True)
    pcnt3 = pcnt[..., None]
    h = _dense(psum1, pcnt3, x_pad, W1l.T, W1r.T, b1l.reshape(1, -1),
               relu=True)
    psum2 = _sc_aggregate(h, src1d, dst1d, zeros, ones, iota,
                          with_cnt=False)[0]
    out = _dense(psum2, pcnt3, h, W2l.T, W2r.T, b2l.reshape(1, -1),
                 relu=False)
    return out[:n]


# speedup vs baseline: 1.3182x; 1.3182x over previous
import functools

import jax
import jax.numpy as jnp
from jax import lax
from jax.experimental import pallas as pl
from jax.experimental.pallas import tpu as pltpu
from jax.experimental.pallas import tpu_sc as plsc

_NC = 2
_NS = 16
_CH = 128
_NZ = 5
_NPAD = _NS * _NZ * _CH


def _sc_aggregate(feat, src1d, dst1d, zeros_hbm, ones_hbm, iota_hbm,
                  *, with_cnt):
    n_pad, d = feat.shape
    assert n_pad == _NPAD
    e_pad = src1d.shape[0]
    cpt = e_pad // (_NC * _NS * _CH)
    assert e_pad == cpt * _NC * _NS * _CH

    out_type = [jax.ShapeDtypeStruct((_NC, n_pad, d), jnp.float32)]
    if with_cnt:
        out_type.append(jax.ShapeDtypeStruct((_NC, n_pad), jnp.float32))

    scratch = [
        pltpu.VMEM((_CH,), jnp.int32),
        pltpu.VMEM((_CH,), jnp.int32),
        pltpu.VMEM((_CH, d), jnp.float32),
        pltpu.VMEM((_CH,), jnp.float32),
        pltpu.VMEM_SHARED((n_pad, d), jnp.float32),
        pltpu.SemaphoreType.DMA,
    ]
    if with_cnt:
        scratch.append(pltpu.VMEM_SHARED((n_pad,), jnp.float32))

    def body(feat_hbm, src_hbm, dst_hbm, z_hbm, o_hbm, iota_hbm,
             psum_hbm, *rest):
        if with_cnt:
            pcnt_hbm, src_v, dst_v, rows_v, c_v, acc, sem, cacc = rest
        else:
            src_v, dst_v, rows_v, c_v, acc, sem = rest

        cid = lax.axis_index("c")
        sid = lax.axis_index("s")
        wid = cid * _NS + sid

        pltpu.sync_copy(z_hbm, rows_v)
        pltpu.sync_copy(z_hbm.at[0], c_v)
        zbase = sid * (_NZ * _CH)
        for k in range(_NZ):
            pltpu.sync_copy(iota_hbm.at[pl.ds(zbase + k * _CH, _CH)], src_v)
            pltpu.sync_copy(rows_v, acc.at[src_v])
            if with_cnt:
                pltpu.sync_copy(c_v, cacc.at[src_v])
        if with_cnt:
            pltpu.sync_copy(o_hbm, c_v)
        plsc.subcore_barrier()

        def step(j, carry):
            off = (wid * cpt + j) * _CH
            pltpu.sync_copy(src_hbm.at[pl.ds(off, _CH)], src_v)
            pltpu.sync_copy(dst_hbm.at[pl.ds(off, _CH)], dst_v)
            pltpu.async_copy(feat_hbm.at[src_v], rows_v, sem).wait()
            pltpu.sync_copy(rows_v, acc.at[dst_v], add=True)
            if with_cnt:
                pltpu.sync_copy(c_v, cacc.at[dst_v], add=True)
            return carry

        lax.fori_loop(0, cpt, step, 0)
        plsc.subcore_barrier()

        for k in range(_NZ):
            base = zbase + k * _CH
            pltpu.sync_copy(iota_hbm.at[pl.ds(base, _CH)], src_v)
            pltpu.async_copy(acc.at[src_v], rows_v, sem).wait()
            pltpu.sync_copy(rows_v, psum_hbm.at[cid, pl.ds(base, _CH)])
            if with_cnt:
                pltpu.async_copy(cacc.at[src_v], c_v, sem).wait()
                pltpu.sync_copy(c_v, pcnt_hbm.at[cid, pl.ds(base, _CH)])

    mesh = plsc.VectorSubcoreMesh(core_axis_name="c", subcore_axis_name="s")
    fn = pl.kernel(body, out_type=out_type, mesh=mesh, scratch_types=scratch)
    return fn(feat, src1d, dst1d, zeros_hbm, ones_hbm, iota_hbm)


def _dense_body(relu, psum_ref, cnt_ref, x_ref, wl_ref, wr_ref, b_ref, o_ref):
    s = psum_ref[0] + psum_ref[1]
    c = cnt_ref[0] + cnt_ref[1]
    agg = s * (1.0 / jnp.maximum(c, 1.0))
    y = jnp.dot(agg, wl_ref[...], preferred_element_type=jnp.float32)
    y = y + jnp.dot(x_ref[...], wr_ref[...], preferred_element_type=jnp.float32)
    y = y + b_ref[...]
    o_ref[...] = jnp.maximum(y, 0.0) if relu else y


def _dense(psum, pcnt3, x, wlT, wrT, b, *, relu, block_rows=512):
    n, d = x.shape
    hid = wlT.shape[1]
    assert n % block_rows == 0
    grid = (n // block_rows,)
    return pl.pallas_call(
        functools.partial(_dense_body, relu),
        grid=grid,
        in_specs=[
            pl.BlockSpec((_NC, block_rows, d), lambda i: (0, i, 0)),
            pl.BlockSpec((_NC, block_rows, 1), lambda i: (0, i, 0)),
            pl.BlockSpec((block_rows, d), lambda i: (i, 0)),
            pl.BlockSpec((d, hid), lambda i: (0, 0)),
            pl.BlockSpec((d, hid), lambda i: (0, 0)),
            pl.BlockSpec((1, hid), lambda i: (0, 0)),
        ],
        out_specs=pl.BlockSpec((block_rows, hid), lambda i: (i, 0)),
        out_shape=jax.ShapeDtypeStruct((n, hid), jnp.float32),
    )(psum, pcnt3, x, wlT, wrT, b)


def kernel(x, edge_index, W1l, b1l, W1r, W2l, b2l, W2r):
    n, d = x.shape
    e = edge_index.shape[1]
    assert n <= _NPAD

    e_pad = -(-e // (_NC * _NS * _CH)) * (_NC * _NS * _CH)
    src1d = jnp.concatenate(
        [edge_index[0], jnp.zeros((e_pad - e,), jnp.int32)])
    dst1d = jnp.concatenate(
        [edge_index[1], jnp.full((e_pad - e,), n, jnp.int32)])
    zeros = jnp.zeros((_CH, d), jnp.float32)
    ones = jnp.ones((_CH,), jnp.float32)
    iota = jnp.arange(_NPAD, dtype=jnp.int32)
    x_pad = jnp.concatenate(
        [x, jnp.zeros((_NPAD - n, d), jnp.float32)], axis=0)

    psum1, pcnt = _sc_aggregate(x_pad, src1d, dst1d, zeros, ones, iota,
                                with_cnt=True)
    pcnt3 = pcnt[..., None]
    h = _dense(psum1, pcnt3, x_pad, W1l.T, W1r.T, b1l.reshape(1, -1),
               relu=True)
    psum2 = _sc_aggregate(h, src1d, dst1d, zeros, ones, iota,
                          with_cnt=False)[0]
    out = _dense(psum2, pcnt3, h, W2l.T, W2r.T, b2l.reshape(1, -1),
                 relu=False)
    return out[:n]


# speedup vs baseline: 1.4214x; 1.0783x over previous
import functools

import jax
import jax.numpy as jnp
from jax import lax
from jax.experimental import pallas as pl
from jax.experimental.pallas import tpu as pltpu
from jax.experimental.pallas import tpu_sc as plsc

_NC = 2
_NS = 16
_CH = 128
_NZ = 5
_NPAD = _NS * _NZ * _CH


def _sc_aggregate(feat, src1d, dst1d, zeros_hbm, ones_hbm, iota_hbm,
                  *, with_cnt):
    n_pad, d = feat.shape
    assert n_pad == _NPAD
    e_pad = src1d.shape[0]
    cpt = e_pad // (_NC * _NS * _CH)
    assert e_pad == cpt * _NC * _NS * _CH

    out_type = [jax.ShapeDtypeStruct((_NC, n_pad, d), jnp.float32)]
    if with_cnt:
        out_type.append(jax.ShapeDtypeStruct((_NC, n_pad), jnp.float32))

    scratch = [
        pltpu.VMEM((_CH,), jnp.int32),
        pltpu.VMEM((_CH,), jnp.int32),
        pltpu.VMEM((_CH, d), jnp.float32),
        pltpu.VMEM((_CH,), jnp.float32),
        pltpu.VMEM_SHARED((n_pad, d), jnp.float32),
        pltpu.SemaphoreType.DMA,
        pltpu.SemaphoreType.DMA,
    ]
    if with_cnt:
        scratch.append(pltpu.VMEM_SHARED((n_pad,), jnp.float32))

    def body(feat_hbm, src_hbm, dst_hbm, z_hbm, o_hbm, iota_hbm,
             psum_hbm, *rest):
        if with_cnt:
            pcnt_hbm, src_v, dst_v, rows_v, c_v, acc, sem, semi, cacc = rest
        else:
            src_v, dst_v, rows_v, c_v, acc, sem, semi = rest

        cid = lax.axis_index("c")
        sid = lax.axis_index("s")
        wid = cid * _NS + sid

        pltpu.sync_copy(z_hbm, rows_v)
        pltpu.sync_copy(z_hbm.at[0], c_v)
        zbase = sid * (_NZ * _CH)
        for k in range(_NZ):
            pltpu.sync_copy(iota_hbm.at[pl.ds(zbase + k * _CH, _CH)], src_v)
            pltpu.sync_copy(rows_v, acc.at[src_v])
            if with_cnt:
                pltpu.sync_copy(c_v, cacc.at[src_v])
        if with_cnt:
            pltpu.sync_copy(o_hbm, c_v)
        plsc.subcore_barrier()

        def step(j, carry):
            off = (wid * cpt + j) * _CH
            dcp = pltpu.async_copy(dst_hbm.at[pl.ds(off, _CH)], dst_v, semi)
            pltpu.sync_copy(src_hbm.at[pl.ds(off, _CH)], src_v)
            pltpu.async_copy(feat_hbm.at[src_v], rows_v, sem).wait()
            dcp.wait()
            pltpu.sync_copy(rows_v, acc.at[dst_v], add=True)
            if with_cnt:
                pltpu.sync_copy(c_v, cacc.at[dst_v], add=True)
            return carry

        lax.fori_loop(0, cpt, step, 0)
        plsc.subcore_barrier()

        for k in range(_NZ):
            base = zbase + k * _CH
            pltpu.sync_copy(iota_hbm.at[pl.ds(base, _CH)], src_v)
            pltpu.async_copy(acc.at[src_v], rows_v, sem).wait()
            pltpu.sync_copy(rows_v, psum_hbm.at[cid, pl.ds(base, _CH)])
            if with_cnt:
                pltpu.async_copy(cacc.at[src_v], c_v, sem).wait()
                pltpu.sync_copy(c_v, pcnt_hbm.at[cid, pl.ds(base, _CH)])

    mesh = plsc.VectorSubcoreMesh(core_axis_name="c", subcore_axis_name="s")
    fn = pl.kernel(body, out_type=out_type, mesh=mesh, scratch_types=scratch)
    return fn(feat, src1d, dst1d, zeros_hbm, ones_hbm, iota_hbm)


def _dense_body(relu, psum_ref, cnt_ref, x_ref, wl_ref, wr_ref, b_ref, o_ref):
    s = psum_ref[0] + psum_ref[1]
    c = cnt_ref[0] + cnt_ref[1]
    agg = s * (1.0 / jnp.maximum(c, 1.0))
    y = jnp.dot(agg, wl_ref[...], preferred_element_type=jnp.float32)
    y = y + jnp.dot(x_ref[...], wr_ref[...], preferred_element_type=jnp.float32)
    y = y + b_ref[...]
    o_ref[...] = jnp.maximum(y, 0.0) if relu else y


def _dense(psum, pcnt3, x, wlT, wrT, b, *, relu, block_rows=512):
    n, d = x.shape
    hid = wlT.shape[1]
    assert n % block_rows == 0
    grid = (n // block_rows,)
    return pl.pallas_call(
        functools.partial(_dense_body, relu),
        grid=grid,
        in_specs=[
            pl.BlockSpec((_NC, block_rows, d), lambda i: (0, i, 0)),
            pl.BlockSpec((_NC, block_rows, 1), lambda i: (0, i, 0)),
            pl.BlockSpec((block_rows, d), lambda i: (i, 0)),
            pl.BlockSpec((d, hid), lambda i: (0, 0)),
            pl.BlockSpec((d, hid), lambda i: (0, 0)),
            pl.BlockSpec((1, hid), lambda i: (0, 0)),
        ],
        out_specs=pl.BlockSpec((block_rows, hid), lambda i: (i, 0)),
        out_shape=jax.ShapeDtypeStruct((n, hid), jnp.float32),
    )(psum, pcnt3, x, wlT, wrT, b)


def kernel(x, edge_index, W1l, b1l, W1r, W2l, b2l, W2r):
    n, d = x.shape
    e = edge_index.shape[1]
    assert n <= _NPAD

    e_pad = -(-e // (_NC * _NS * _CH)) * (_NC * _NS * _CH)
    src1d = jnp.concatenate(
        [edge_index[0], jnp.zeros((e_pad - e,), jnp.int32)])
    dst1d = jnp.concatenate(
        [edge_index[1], jnp.full((e_pad - e,), n, jnp.int32)])
    zeros = jnp.zeros((_CH, d), jnp.float32)
    ones = jnp.ones((_CH,), jnp.float32)
    iota = jnp.arange(_NPAD, dtype=jnp.int32)
    x_pad = jnp.concatenate(
        [x, jnp.zeros((_NPAD - n, d), jnp.float32)], axis=0)

    psum1, pcnt = _sc_aggregate(x_pad, src1d, dst1d, zeros, ones, iota,
                                with_cnt=True)
    pcnt3 = pcnt[..., None]
    h = _dense(psum1, pcnt3, x_pad, W1l.T, W1r.T, b1l.reshape(1, -1),
               relu=True)
    psum2 = _sc_aggregate(h, src1d, dst1d, zeros, ones, iota,
                          with_cnt=False)[0]
    out = _dense(psum2, pcnt3, h, W2l.T, W2r.T, b2l.reshape(1, -1),
                 relu=False)
    return out[:n]


# speedup vs baseline: 1.4297x; 1.0059x over previous
import functools

import jax
import jax.numpy as jnp
from jax import lax
from jax.experimental import pallas as pl
from jax.experimental.pallas import tpu as pltpu
from jax.experimental.pallas import tpu_sc as plsc

_NC = 2
_NS = 16
_CH = 128
_NZ = 5
_NPAD = _NS * _NZ * _CH


def _sc_aggregate(feat, src1d, dst1d, zeros_hbm, ones_hbm, iota_hbm,
                  *, with_cnt):
    n_pad, d = feat.shape
    assert n_pad == _NPAD
    e_pad = src1d.shape[0]
    cpt = e_pad // (_NC * _NS * _CH)
    assert e_pad == cpt * _NC * _NS * _CH

    out_type = [jax.ShapeDtypeStruct((_NC, n_pad, d), jnp.float32)]
    if with_cnt:
        out_type.append(jax.ShapeDtypeStruct((_NC, n_pad), jnp.float32))

    scratch = [
        pltpu.VMEM((_CH,), jnp.int32),
        pltpu.VMEM((_CH,), jnp.int32),
        pltpu.VMEM((_CH, d), jnp.float32),
        pltpu.VMEM((_CH,), jnp.float32),
        pltpu.VMEM_SHARED((n_pad, d), jnp.float32),
        pltpu.SemaphoreType.DMA,
        pltpu.SemaphoreType.DMA,
    ]
    if with_cnt:
        scratch.append(pltpu.VMEM_SHARED((n_pad,), jnp.float32))

    def body(feat_hbm, src_hbm, dst_hbm, z_hbm, o_hbm, iota_hbm,
             psum_hbm, *rest):
        if with_cnt:
            pcnt_hbm, src_v, dst_v, rows_v, c_v, acc, sem, semi, cacc = rest
        else:
            src_v, dst_v, rows_v, c_v, acc, sem, semi = rest

        cid = lax.axis_index("c")
        sid = lax.axis_index("s")
        wid = cid * _NS + sid

        pltpu.sync_copy(z_hbm, rows_v)
        pltpu.sync_copy(z_hbm.at[0], c_v)
        zbase = sid * (_NZ * _CH)
        for k in range(_NZ):
            pltpu.sync_copy(iota_hbm.at[pl.ds(zbase + k * _CH, _CH)], src_v)
            pltpu.sync_copy(rows_v, acc.at[src_v])
            if with_cnt:
                pltpu.sync_copy(c_v, cacc.at[src_v])
        if with_cnt:
            pltpu.sync_copy(o_hbm, c_v)
        plsc.subcore_barrier()

        def step(j, carry):
            off = (wid * cpt + j) * _CH
            dcp = pltpu.async_copy(dst_hbm.at[pl.ds(off, _CH)], dst_v, semi)
            pltpu.sync_copy(src_hbm.at[pl.ds(off, _CH)], src_v)
            pltpu.async_copy(feat_hbm.at[src_v], rows_v, sem).wait()
            dcp.wait()
            if with_cnt:
                ccp = pltpu.async_copy(c_v, cacc.at[dst_v], semi, add=True)
            pltpu.sync_copy(rows_v, acc.at[dst_v], add=True)
            if with_cnt:
                ccp.wait()
            return carry

        lax.fori_loop(0, cpt, step, 0)
        plsc.subcore_barrier()

        for k in range(_NZ):
            base = zbase + k * _CH
            pltpu.sync_copy(iota_hbm.at[pl.ds(base, _CH)], src_v)
            pltpu.async_copy(acc.at[src_v], rows_v, sem).wait()
            pltpu.sync_copy(rows_v, psum_hbm.at[cid, pl.ds(base, _CH)])
            if with_cnt:
                pltpu.async_copy(cacc.at[src_v], c_v, sem).wait()
                pltpu.sync_copy(c_v, pcnt_hbm.at[cid, pl.ds(base, _CH)])

    mesh = plsc.VectorSubcoreMesh(core_axis_name="c", subcore_axis_name="s")
    fn = pl.kernel(body, out_type=out_type, mesh=mesh, scratch_types=scratch)
    return fn(feat, src1d, dst1d, zeros_hbm, ones_hbm, iota_hbm)


def _dense_body(relu, psum_ref, cnt_ref, x_ref, wl_ref, wr_ref, b_ref, o_ref):
    s = psum_ref[0] + psum_ref[1]
    c = cnt_ref[0] + cnt_ref[1]
    agg = s * (1.0 / jnp.maximum(c, 1.0))
    y = jnp.dot(agg, wl_ref[...], preferred_element_type=jnp.float32)
    y = y + jnp.dot(x_ref[...], wr_ref[...], preferred_element_type=jnp.float32)
    y = y + b_ref[...]
    o_ref[...] = jnp.maximum(y, 0.0) if relu else y


def _dense(psum, pcnt3, x, wlT, wrT, b, *, relu, block_rows=512):
    n, d = x.shape
    hid = wlT.shape[1]
    assert n % block_rows == 0
    grid = (n // block_rows,)
    return pl.pallas_call(
        functools.partial(_dense_body, relu),
        grid=grid,
        in_specs=[
            pl.BlockSpec((_NC, block_rows, d), lambda i: (0, i, 0)),
            pl.BlockSpec((_NC, block_rows, 1), lambda i: (0, i, 0)),
            pl.BlockSpec((block_rows, d), lambda i: (i, 0)),
            pl.BlockSpec((d, hid), lambda i: (0, 0)),
            pl.BlockSpec((d, hid), lambda i: (0, 0)),
            pl.BlockSpec((1, hid), lambda i: (0, 0)),
        ],
        out_specs=pl.BlockSpec((block_rows, hid), lambda i: (i, 0)),
        out_shape=jax.ShapeDtypeStruct((n, hid), jnp.float32),
    )(psum, pcnt3, x, wlT, wrT, b)


def kernel(x, edge_index, W1l, b1l, W1r, W2l, b2l, W2r):
    n, d = x.shape
    e = edge_index.shape[1]
    assert n <= _NPAD

    e_pad = -(-e // (_NC * _NS * _CH)) * (_NC * _NS * _CH)
    src1d = jnp.concatenate(
        [edge_index[0], jnp.zeros((e_pad - e,), jnp.int32)])
    dst1d = jnp.concatenate(
        [edge_index[1], jnp.full((e_pad - e,), n, jnp.int32)])
    zeros = jnp.zeros((_CH, d), jnp.float32)
    ones = jnp.ones((_CH,), jnp.float32)
    iota = jnp.arange(_NPAD, dtype=jnp.int32)
    x_pad = jnp.concatenate(
        [x, jnp.zeros((_NPAD - n, d), jnp.float32)], axis=0)

    psum1, pcnt = _sc_aggregate(x_pad, src1d, dst1d, zeros, ones, iota,
                                with_cnt=True)
    pcnt3 = pcnt[..., None]
    h = _dense(psum1, pcnt3, x_pad, W1l.T, W1r.T, b1l.reshape(1, -1),
               relu=True)
    psum2 = _sc_aggregate(h, src1d, dst1d, zeros, ones, iota,
                          with_cnt=False)[0]
    out = _dense(psum2, pcnt3, h, W2l.T, W2r.T, b2l.reshape(1, -1),
                 relu=False)
    return out[:n]
